# baseline (speedup 1.0000x reference)
import functools

import jax
import jax.numpy as jnp
from jax import lax
from jax.experimental import pallas as pl
from jax.experimental.pallas import tpu as pltpu
from jax.experimental.pallas import tpu_sc as plsc

_K = 2000
_TH = 0.001
_NIMG = 12
_H = 512
_W = 512
_NPIX = _H * _W

_GRP = 8
_CHUNK = _NPIX // _GRP
_NVEC = _CHUNK // 16
_NB1 = 2048
_NB2 = 2048
_NB3 = 512


def _f32_val(xi):
    v = jnp.full((8, 128), xi, jnp.int32)
    return jnp.max(lax.bitcast_convert_type(v, jnp.float32))


def _nmin4(x):
    a = pltpu.roll(x, 1, 0)
    b = pltpu.roll(x, _H - 1, 0)
    c = pltpu.roll(x, 1, 1)
    d = pltpu.roll(x, _W - 1, 1)
    return jnp.minimum(jnp.minimum(a, b), jnp.minimum(c, d))


def _prep_body(p_ref, yt_ref, xb0_ref, xb1_ref, stats_ref, cnt_ref):
    i = pl.program_id(0)
    ch = i % 3 + 1
    p = p_ref[0, 0]
    mn = jnp.min(p)
    x0 = p - mn
    xb0_ref[0] = lax.bitcast_convert_type(x0, jnp.int32)
    x1 = jnp.maximum(p - _nmin4(p), 0.0)
    xb1_ref[0] = lax.bitcast_convert_type(x1, jnp.int32)
    stats_ref[0, i] = jnp.max(p) - mn
    stats_ref[1, i] = jnp.max(x1)
    stats_ref[2, i] = mn

    yt = yt_ref[0]
    e = (yt == ch).astype(jnp.int32)
    n1 = jnp.sum(e)
    c0 = jnp.where(n1 >= _NPIX, 0, jnp.minimum(n1, _K))
    nbmin = _nmin4(e)
    eb = e * (1 - nbmin)
    c1 = jnp.minimum(jnp.sum(eb), _K)
    cnt_ref[0, i] = c0
    cnt_ref[1, i] = c1


def _prep(y_pred_softmax, y_true):
    return pl.pallas_call(
        _prep_body,
        grid=(_NIMG,),
        in_specs=[
            pl.BlockSpec((1, 1, _H, _W), lambda i: (i // 3, i % 3 + 1, 0, 0)),
            pl.BlockSpec((1, _H, _W), lambda i: (i // 3, 0, 0)),
        ],
        out_specs=[
            pl.BlockSpec((1, _H, _W), lambda i: (i, 0, 0)),
            pl.BlockSpec((1, _H, _W), lambda i: (i, 0, 0)),
            pl.BlockSpec(memory_space=pltpu.SMEM),
            pl.BlockSpec(memory_space=pltpu.SMEM),
        ],
        out_shape=[
            jax.ShapeDtypeStruct((_NIMG, _H, _W), jnp.int32),
            jax.ShapeDtypeStruct((_NIMG, _H, _W), jnp.int32),
            jax.ShapeDtypeStruct((3, _NIMG), jnp.float32),
            jax.ShapeDtypeStruct((2, _NIMG), jnp.int32),
        ],
    )(y_pred_softmax, y_true)


def _zero(h, nb):
    z = jnp.zeros((16,), jnp.int32)

    def zb(j, _):
        h[pl.ds(j * 16, 16)] = z
        return 0

    lax.fori_loop(0, nb // 16, zb, 0)


def _swizzle(b, rlog):
    return ((b & ((1 << rlog) - 1)) << 4) | (b >> rlog)


def _scan1(chunk, h, rlog):
    ones = jnp.ones((16,), jnp.int32)

    def b1(j, _):
        x = chunk[pl.ds(j * 16, 16)]
        plsc.addupdate_scatter(h, [_swizzle(x >> 20, rlog)], ones)
        return 0

    lax.fori_loop(0, _NVEC, b1, 0)


def _scan_masked(chunk, h, keysh, key, bsh, bmask, rlog):
    ones = jnp.ones((16,), jnp.int32)

    def b(j, _):
        x = chunk[pl.ds(j * 16, 16)]
        m = (x >> keysh) == key
        plsc.addupdate_scatter(h, [_swizzle((x >> bsh) & bmask, rlog)],
                               ones, mask=m)
        return 0

    lax.fori_loop(0, _NVEC, b, 0)


def _hsum(v):
    acc = v[0]
    for l in range(1, 16):
        acc = acc + v[l]
    return acc


def _combine(h, stage, scomb, part, tmp, comb, g, t, nb):
    sl = nb // _GRP
    pltpu.sync_copy(h.at[pl.ds(0, nb)], stage.at[g, t, pl.ds(0, nb)])
    plsc.subcore_barrier()
    pltpu.sync_copy(stage.at[g, :, pl.ds(t * sl, sl)], part.at[:, pl.ds(0, sl)])

    def cb(jv, _):
        acc = part[0, pl.ds(jv * 16, 16)]
        for rr in range(1, _GRP):
            acc = acc + part[rr, pl.ds(jv * 16, 16)]
        tmp[pl.ds(jv * 16, 16)] = acc
        return 0

    lax.fori_loop(0, sl // 16, cb, 0)
    pltpu.sync_copy(tmp.at[pl.ds(0, sl)], scomb.at[g, pl.ds(t * sl, sl)])
    plsc.subcore_barrier()
    pltpu.sync_copy(scomb.at[g, pl.ds(0, nb)], comb.at[pl.ds(0, nb)])


def _search(comb, nb, r):
    rows = nb // 16
    z = jnp.zeros((16,), jnp.int32)
    iota = lax.iota(jnp.int32, 16)

    def tb(j, acc):
        return acc + comb[pl.ds(j * 16, 16)]

    tvec = lax.fori_loop(0, rows, tb, z)
    total = _hsum(tvec)
    thr = total - r
    base = z
    acc = jnp.int32(0)
    for l in range(16):
        base = jnp.where(iota == l, acc, base)
        acc = acc + tvec[l]

    def sb(j, carry):
        pref, cnt = carry
        v = comb[pl.ds(j * 16, 16)]
        cnt = cnt + jnp.where(base + pref <= thr, 1, 0)
        return pref + v, cnt

    _, cntv = lax.fori_loop(0, rows, sb, (z, z))
    b = _hsum(cntv) - 1

    def ab(j, av):
        v = comb[pl.ds(j * 16, 16)]
        idx = iota * rows + j
        return av + jnp.where(idx > b, v, 0)

    A = _hsum(lax.fori_loop(0, rows, ab, z))
    return b, A


def _sc_body(xb, rk, th, chunk, rowv, hist, comb, part, tmp, stage, scomb):
    c = lax.axis_index("c")
    s = lax.axis_index("s")
    g = s // _GRP
    t = s % _GRP

    def round_body(rnd, _):
        a = c * _NIMG + rnd * 2 + g
        pltpu.sync_copy(xb.at[a, pl.ds(t * _CHUNK, _CHUNK)], chunk)
        pltpu.sync_copy(rk.at[a], rowv)
        r_c = rowv[...][0]

        _zero(hist, _NB1)
        _scan1(chunk, hist, 7)
        _combine(hist, stage, scomb, part, tmp, comb, g, t, _NB1)
        bk, ak = _search(comb, _NB1, jnp.int32(_K))
        bc, ac = _search(comb, _NB1, r_c)
        neq = r_c != _K

        _zero(hist, _NB2)
        _scan_masked(chunk, hist, 20, bk, 9, _NB2 - 1, 7)
        _combine(hist, stage, scomb, part, tmp, comb, g, t, _NB2)
        b2k, a2k = _search(comb, _NB2, _K - ak)
        _zero(hist, _NB3)
        _scan_masked(chunk, hist, 9, (bk << 11) | b2k, 0, _NB3 - 1, 5)
        _combine(hist, stage, scomb, part, tmp, comb, g, t, _NB3)
        b3k, _u = _search(comb, _NB3, _K - ak - a2k)
        lo_k = (bk << 20) | (b2k << 9) | b3k

        _zero(hist, _NB2)

        @pl.when(neq)
        def _():
            _scan_masked(chunk, hist, 20, bc, 9, _NB2 - 1, 7)

        _combine(hist, stage, scomb, part, tmp, comb, g, t, _NB2)
        b2c, a2c = _search(comb, _NB2, r_c - ac)
        _zero(hist, _NB3)

        @pl.when(neq)
        def _():
            _scan_masked(chunk, hist, 9, (bc << 11) | b2c, 0, _NB3 - 1, 5)

        _combine(hist, stage, scomb, part, tmp, comb, g, t, _NB3)
        b3c, _u2 = _search(comb, _NB3, r_c - ac - a2c)
        lo_c = jnp.where(neq, (bc << 20) | (b2c << 9) | b3c, lo_k)

        @pl.when(t == 0)
        def _():
            iota = lax.iota(jnp.int32, 16)
            rowv[...] = jnp.where(iota == 0, lo_k,
                                  jnp.where(iota == 1, lo_c, 0))
            pltpu.sync_copy(rowv, th.at[a])

        return 0

    lax.fori_loop(0, _NIMG // 2, round_body, 0)


_sc_select = functools.partial(
    pl.kernel,
    out_type=jax.ShapeDtypeStruct((2 * _NIMG, 16), jnp.int32),
    mesh=plsc.VectorSubcoreMesh(core_axis_name="c", subcore_axis_name="s"),
    scratch_types=[
        pltpu.VMEM((_CHUNK,), jnp.int32),
        pltpu.VMEM((16,), jnp.int32),
        pltpu.VMEM((_NB1,), jnp.int32),
        pltpu.VMEM((_NB1,), jnp.int32),
        pltpu.VMEM((_GRP, _NB1 // _GRP), jnp.int32),
        pltpu.VMEM((_NB1 // _GRP,), jnp.int32),
        pltpu.VMEM_SHARED((2, _GRP, _NB1), jnp.int32),
        pltpu.VMEM_SHARED((2, _NB1), jnp.int32),
    ],
    compiler_params=pltpu.CompilerParams(needs_layout_passes=False),
)(_sc_body)


def _final_body(p_ref, stats_ref, cnt_ref, th_ref, out_ref, acc_ref):
    i = pl.program_id(0)
    p = p_ref[0, 0]

    @pl.when(i == 0)
    def _init():
        acc_ref[0] = 0.0
        acc_ref[1] = 0.0

    for v in (0, 1):
        if v == 0:
            x = p - stats_ref[2, i]
        else:
            x = jnp.maximum(p - _nmin4(p), 0.0)
        gmax = stats_ref[v, 0]
        for j in range(1, _NIMG):
            gmax = jnp.maximum(gmax, stats_ref[v, j])
        g = jnp.maximum(gmax, 1e-8)
        tau = _TH * g
        c = cnt_ref[v, i]
        tk = _f32_val(th_ref[v * _NIMG + i, 0])
        tc = _f32_val(th_ref[v * _NIMG + i, 1])

        m2 = jnp.maximum(tk, tau)
        m1 = jnp.maximum(tc, tau)
        mask2 = x > m2
        mask1 = x > m1
        a2 = jnp.sum(jnp.where(mask2, x * x, 0.0))
        c2 = jnp.sum(mask2.astype(jnp.int32)).astype(jnp.float32)
        a1 = jnp.sum(jnp.where(mask1, x, 0.0))
        c1n = jnp.sum(mask1.astype(jnp.int32)).astype(jnp.float32)
        cf = c.astype(jnp.float32)
        s2 = (a2 + jnp.where(tk > tau, (_K - c2) * tk * tk, 0.0)) / (g * g)
        t_sum = jnp.where(
            c > 0,
            (a1 + jnp.where(tc > tau, (cf - c1n) * tc, 0.0)) / g,
            0.0)
        acc_ref[v] = acc_ref[v] + s2 - 2.0 * t_sum + cf

    @pl.when(i == _NIMG - 1)
    def _final():
        l0 = acc_ref[0] / (_NIMG * _K)
        l1 = acc_ref[1] / (_NIMG * _K)
        den = jnp.maximum(l0 + l1, 1e-8)
        out_ref[...] = jnp.full((1, 1), 2.0 * l0 * l1 / den, jnp.float32)


def _final(y_pred_softmax, stats, cnt, th):
    return pl.pallas_call(
        _final_body,
        grid=(_NIMG,),
        in_specs=[
            pl.BlockSpec((1, 1, _H, _W), lambda i: (i // 3, i % 3 + 1, 0, 0)),
            pl.BlockSpec(memory_space=pltpu.SMEM),
            pl.BlockSpec(memory_space=pltpu.SMEM),
            pl.BlockSpec(memory_space=pltpu.SMEM),
        ],
        out_specs=pl.BlockSpec((1, 1), lambda i: (0, 0)),
        out_shape=jax.ShapeDtypeStruct((1, 1), jnp.float32),
        scratch_shapes=[pltpu.SMEM((2,), jnp.float32)],
    )(y_pred_softmax, stats, cnt, th)


def kernel(y_pred_softmax, y_true):
    xb0, xb1, stats, cnt = _prep(y_pred_softmax, y_true)
    xb = jnp.concatenate(
        [xb0.reshape(_NIMG, _NPIX), xb1.reshape(_NIMG, _NPIX)], axis=0)
    ranks = jnp.clip(cnt, 1, _K).reshape(2 * _NIMG).astype(jnp.int32)
    rank_pad = jnp.zeros((2 * _NIMG, 16), jnp.int32).at[:, 0].set(ranks)
    th = _sc_select(xb, rank_pad)
    return _final(y_pred_softmax, stats, cnt, th)[0, 0]


# speedup vs baseline: 1.0890x; 1.0890x over previous
import functools

import jax
import jax.numpy as jnp
from jax import lax
from jax.experimental import pallas as pl
from jax.experimental.pallas import tpu as pltpu
from jax.experimental.pallas import tpu_sc as plsc

_K = 2000
_TH = 0.001
_NIMG = 12
_H = 512
_W = 512
_NPIX = _H * _W

_GRP = 8
_CHUNK = _NPIX // _GRP
_NVEC = _CHUNK // 16
_NB1 = 2048
_NB2 = 2048
_NB3 = 512


def _f32_val(xi):
    v = jnp.full((8, 128), xi, jnp.int32)
    return jnp.max(lax.bitcast_convert_type(v, jnp.float32))


def _nmin4(x):
    a = pltpu.roll(x, 1, 0)
    b = pltpu.roll(x, _H - 1, 0)
    c = pltpu.roll(x, 1, 1)
    d = pltpu.roll(x, _W - 1, 1)
    return jnp.minimum(jnp.minimum(a, b), jnp.minimum(c, d))


def _prep_body(p_ref, yt_ref, xb0_ref, xb1_ref, stats_ref, cnt_ref):
    i = pl.program_id(0)
    ch = i % 3 + 1
    p = p_ref[0, 0]
    mn = jnp.min(p)
    x0 = p - mn
    xb0_ref[0] = lax.bitcast_convert_type(x0, jnp.int32)
    x1 = jnp.maximum(p - _nmin4(p), 0.0)
    xb1_ref[0] = lax.bitcast_convert_type(x1, jnp.int32)
    stats_ref[0, i] = jnp.max(p) - mn
    stats_ref[1, i] = jnp.max(x1)
    stats_ref[2, i] = mn

    yt = yt_ref[0]
    e = (yt == ch).astype(jnp.int32)
    n1 = jnp.sum(e)
    c0 = jnp.where(n1 >= _NPIX, 0, jnp.minimum(n1, _K))
    nbmin = _nmin4(e)
    eb = e * (1 - nbmin)
    c1 = jnp.minimum(jnp.sum(eb), _K)
    cnt_ref[0, i] = c0
    cnt_ref[1, i] = c1


def _prep(y_pred_softmax, y_true):
    return pl.pallas_call(
        _prep_body,
        grid=(_NIMG,),
        in_specs=[
            pl.BlockSpec((1, 1, _H, _W), lambda i: (i // 3, i % 3 + 1, 0, 0)),
            pl.BlockSpec((1, _H, _W), lambda i: (i // 3, 0, 0)),
        ],
        out_specs=[
            pl.BlockSpec((1, _H, _W), lambda i: (i, 0, 0)),
            pl.BlockSpec((1, _H, _W), lambda i: (i, 0, 0)),
            pl.BlockSpec(memory_space=pltpu.SMEM),
            pl.BlockSpec(memory_space=pltpu.SMEM),
        ],
        out_shape=[
            jax.ShapeDtypeStruct((_NIMG, _H, _W), jnp.int32),
            jax.ShapeDtypeStruct((_NIMG, _H, _W), jnp.int32),
            jax.ShapeDtypeStruct((3, _NIMG), jnp.float32),
            jax.ShapeDtypeStruct((2, _NIMG), jnp.int32),
        ],
    )(y_pred_softmax, y_true)


def _uloop(n, step, body):
    def b(j, carry):
        for u in range(step):
            carry = body(j * step + u, carry)
        return carry

    return lax.fori_loop(0, n // step, b, 0)


def _zero(h, nb):
    z = jnp.zeros((16,), jnp.int32)

    def zb(j, _):
        h[pl.ds(j * 16, 16)] = z
        return 0

    _uloop(nb // 16, 4, zb)


def _swizzle(b, rlog):
    return ((b & ((1 << rlog) - 1)) << 4) | (b >> rlog)


def _scan1(chunk, h):
    ones = jnp.ones((16,), jnp.int32)

    def b1(j, _):
        x = chunk[pl.ds(j * 16, 16)]
        plsc.addupdate_scatter(h, [_swizzle(x >> 20, 7)], ones)
        return 0

    _uloop(_NVEC, 4, b1)


def _scan_dual(chunk, h, keysh, key_k, key_c, bsh, bmask, rlog, coff):
    ones = jnp.ones((16,), jnp.int32)

    def b(j, _):
        x = chunk[pl.ds(j * 16, 16)]
        key = x >> keysh
        addr = _swizzle((x >> bsh) & bmask, rlog)
        plsc.addupdate_scatter(h, [addr], ones, mask=key == key_k)
        plsc.addupdate_scatter(h, [addr + coff], ones, mask=key == key_c)
        return 0

    _uloop(_NVEC, 4, b)


def _hsum(v):
    acc = v[0]
    for l in range(1, 16):
        acc = acc + v[l]
    return acc


def _combine(h, stage, scomb, part, tmp, comb, g, t, nb):
    sl = nb // _GRP
    pltpu.sync_copy(h.at[pl.ds(0, nb)], stage.at[g, t, pl.ds(0, nb)])
    plsc.subcore_barrier()
    pltpu.sync_copy(stage.at[g, :, pl.ds(t * sl, sl)], part.at[:, pl.ds(0, sl)])

    def cb(jv, _):
        acc = part[0, pl.ds(jv * 16, 16)]
        for rr in range(1, _GRP):
            acc = acc + part[rr, pl.ds(jv * 16, 16)]
        tmp[pl.ds(jv * 16, 16)] = acc
        return 0

    _uloop(sl // 16, 2, cb)
    pltpu.sync_copy(tmp.at[pl.ds(0, sl)], scomb.at[g, pl.ds(t * sl, sl)])
    plsc.subcore_barrier()
    pltpu.sync_copy(scomb.at[g, pl.ds(0, nb)], comb.at[pl.ds(0, nb)])


def _totals(comb, nb, off):
    rows = nb // 16
    z = jnp.zeros((16,), jnp.int32)
    iota = lax.iota(jnp.int32, 16)

    def tb(j, acc):
        return acc + comb[pl.ds(off + j * 16, 16)]

    def tb4(j, acc):
        for u in range(4):
            acc = acc + comb[pl.ds(off + (j * 4 + u) * 16, 16)]
        return acc

    tvec = lax.fori_loop(0, rows // 4, tb4, z)
    total = _hsum(tvec)
    base = z
    acc = jnp.int32(0)
    for l in range(16):
        base = jnp.where(iota == l, acc, base)
        acc = acc + tvec[l]
    return total, base


def _rank_pass(comb, nb, off, total, base, r):
    rows = nb // 16
    z = jnp.zeros((16,), jnp.int32)
    thr = total - r

    def sb(j, carry):
        pref, cnt, ele = carry
        v = comb[pl.ds(off + j * 16, 16)]
        m = base + pref <= thr
        cnt = cnt + jnp.where(m, 1, 0)
        ele = ele + jnp.where(m, v, 0)
        return pref + v, cnt, ele

    def sb2(j, carry):
        for u in range(2):
            carry = sb(j * 2 + u, carry)
        return carry

    _, cntv, elev = lax.fori_loop(0, rows // 2, sb2, (z, z, z))
    b = _hsum(cntv) - 1
    A = total - _hsum(elev)
    return b, A


def _sc_body(xb, rk, th, chunk, rowv, hist, comb, part, tmp, stage, scomb):
    c = lax.axis_index("c")
    s = lax.axis_index("s")
    g = s // _GRP
    t = s % _GRP

    def round_body(rnd, _):
        a = c * _NIMG + rnd * 2 + g
        pltpu.sync_copy(xb.at[a, pl.ds(t * _CHUNK, _CHUNK)], chunk)
        pltpu.sync_copy(rk.at[a], rowv)
        r_c = rowv[...][0]

        _zero(hist, _NB1)
        _scan1(chunk, hist)
        _combine(hist, stage, scomb, part, tmp, comb, g, t, _NB1)
        tot1, base1 = _totals(comb, _NB1, 0)
        bk, ak = _rank_pass(comb, _NB1, 0, tot1, base1, jnp.int32(_K))
        bc, ac = _rank_pass(comb, _NB1, 0, tot1, base1, r_c)

        _zero(hist, 2 * _NB2)
        _scan_dual(chunk, hist, 20, bk, bc, 9, _NB2 - 1, 7, _NB2)
        _combine(hist, stage, scomb, part, tmp, comb, g, t, 2 * _NB2)
        tk2, bsk2 = _totals(comb, _NB2, 0)
        b2k, a2k = _rank_pass(comb, _NB2, 0, tk2, bsk2, _K - ak)
        tc2, bsc2 = _totals(comb, _NB2, _NB2)
        b2c, a2c = _rank_pass(comb, _NB2, _NB2, tc2, bsc2, r_c - ac)

        _zero(hist, 2 * _NB3)
        _scan_dual(chunk, hist, 9, (bk << 11) | b2k, (bc << 11) | b2c,
                   0, _NB3 - 1, 5, _NB3)
        _combine(hist, stage, scomb, part, tmp, comb, g, t, 2 * _NB3)
        tk3, bsk3 = _totals(comb, _NB3, 0)
        b3k, _u = _rank_pass(comb, _NB3, 0, tk3, bsk3, _K - ak - a2k)
        tc3, bsc3 = _totals(comb, _NB3, _NB3)
        b3c, _u2 = _rank_pass(comb, _NB3, _NB3, tc3, bsc3, r_c - ac - a2c)

        lo_k = (bk << 20) | (b2k << 9) | b3k
        lo_c = (bc << 20) | (b2c << 9) | b3c

        @pl.when(t == 0)
        def _():
            iota = lax.iota(jnp.int32, 16)
            rowv[...] = jnp.where(iota == 0, lo_k,
                                  jnp.where(iota == 1, lo_c, 0))
            pltpu.sync_copy(rowv, th.at[a])

        return 0

    lax.fori_loop(0, _NIMG // 2, round_body, 0)


_sc_select = functools.partial(
    pl.kernel,
    out_type=jax.ShapeDtypeStruct((2 * _NIMG, 16), jnp.int32),
    mesh=plsc.VectorSubcoreMesh(core_axis_name="c", subcore_axis_name="s"),
    scratch_types=[
        pltpu.VMEM((_CHUNK,), jnp.int32),
        pltpu.VMEM((16,), jnp.int32),
        pltpu.VMEM((2 * _NB2,), jnp.int32),
        pltpu.VMEM((2 * _NB2,), jnp.int32),
        pltpu.VMEM((_GRP, 2 * _NB2 // _GRP), jnp.int32),
        pltpu.VMEM((2 * _NB2 // _GRP,), jnp.int32),
        pltpu.VMEM_SHARED((2, _GRP, 2 * _NB2), jnp.int32),
        pltpu.VMEM_SHARED((2, 2 * _NB2), jnp.int32),
    ],
    compiler_params=pltpu.CompilerParams(needs_layout_passes=False),
)(_sc_body)


def _final_body(p_ref, stats_ref, cnt_ref, th_ref, out_ref, acc_ref):
    i = pl.program_id(0)
    p = p_ref[0, 0]

    @pl.when(i == 0)
    def _init():
        acc_ref[0] = 0.0
        acc_ref[1] = 0.0

    for v in (0, 1):
        if v == 0:
            x = p - stats_ref[2, i]
        else:
            x = jnp.maximum(p - _nmin4(p), 0.0)
        gmax = stats_ref[v, 0]
        for j in range(1, _NIMG):
            gmax = jnp.maximum(gmax, stats_ref[v, j])
        g = jnp.maximum(gmax, 1e-8)
        tau = _TH * g
        c = cnt_ref[v, i]
        tk = _f32_val(th_ref[v * _NIMG + i, 0])
        tc = _f32_val(th_ref[v * _NIMG + i, 1])

        m2 = jnp.maximum(tk, tau)
        m1 = jnp.maximum(tc, tau)
        mask2 = x > m2
        mask1 = x > m1
        a2 = jnp.sum(jnp.where(mask2, x * x, 0.0))
        c2 = jnp.sum(mask2.astype(jnp.int32)).astype(jnp.float32)
        a1 = jnp.sum(jnp.where(mask1, x, 0.0))
        c1n = jnp.sum(mask1.astype(jnp.int32)).astype(jnp.float32)
        cf = c.astype(jnp.float32)
        s2 = (a2 + jnp.where(tk > tau, (_K - c2) * tk * tk, 0.0)) / (g * g)
        t_sum = jnp.where(
            c > 0,
            (a1 + jnp.where(tc > tau, (cf - c1n) * tc, 0.0)) / g,
            0.0)
        acc_ref[v] = acc_ref[v] + s2 - 2.0 * t_sum + cf

    @pl.when(i == _NIMG - 1)
    def _final():
        l0 = acc_ref[0] / (_NIMG * _K)
        l1 = acc_ref[1] / (_NIMG * _K)
        den = jnp.maximum(l0 + l1, 1e-8)
        out_ref[...] = jnp.full((1, 1), 2.0 * l0 * l1 / den, jnp.float32)


def _final(y_pred_softmax, stats, cnt, th):
    return pl.pallas_call(
        _final_body,
        grid=(_NIMG,),
        in_specs=[
            pl.BlockSpec((1, 1, _H, _W), lambda i: (i // 3, i % 3 + 1, 0, 0)),
            pl.BlockSpec(memory_space=pltpu.SMEM),
            pl.BlockSpec(memory_space=pltpu.SMEM),
            pl.BlockSpec(memory_space=pltpu.SMEM),
        ],
        out_specs=pl.BlockSpec((1, 1), lambda i: (0, 0)),
        out_shape=jax.ShapeDtypeStruct((1, 1), jnp.float32),
        scratch_shapes=[pltpu.SMEM((2,), jnp.float32)],
    )(y_pred_softmax, stats, cnt, th)


def kernel(y_pred_softmax, y_true):
    xb0, xb1, stats, cnt = _prep(y_pred_softmax, y_true)
    xb = jnp.concatenate(
        [xb0.reshape(_NIMG, _NPIX), xb1.reshape(_NIMG, _NPIX)], axis=0)
    ranks = jnp.clip(cnt, 1, _K).reshape(2 * _NIMG).astype(jnp.int32)
    rank_pad = jnp.zeros((2 * _NIMG, 16), jnp.int32).at[:, 0].set(ranks)
    th = _sc_select(xb, rank_pad)
    return _final(y_pred_softmax, stats, cnt, th)[0, 0]


# speedup vs baseline: 1.1461x; 1.0524x over previous
import functools

import jax
import jax.numpy as jnp
from jax import lax
from jax.experimental import pallas as pl
from jax.experimental.pallas import tpu as pltpu
from jax.experimental.pallas import tpu_sc as plsc

_K = 2000
_TH = 0.001
_NIMG = 12
_H = 512
_W = 512
_NPIX = _H * _W

_GRP = 8
_CHUNK = _NPIX // _GRP
_NVEC = _CHUNK // 16
_NB1 = 2048
_NB2 = 2048
_NB3 = 512


def _f32_val(xi):
    v = jnp.full((8, 128), xi, jnp.int32)
    return jnp.max(lax.bitcast_convert_type(v, jnp.float32))


def _nmin4(x):
    a = pltpu.roll(x, 1, 0)
    b = pltpu.roll(x, _H - 1, 0)
    c = pltpu.roll(x, 1, 1)
    d = pltpu.roll(x, _W - 1, 1)
    return jnp.minimum(jnp.minimum(a, b), jnp.minimum(c, d))


def _prep_body(p_ref, yt_ref, xb0_ref, xb1_ref, stats_ref, cnt_ref):
    i = pl.program_id(0)
    ch = i % 3 + 1
    p = p_ref[0, 0]
    mn = jnp.min(p)
    x0 = p - mn
    xb0_ref[0] = lax.bitcast_convert_type(x0, jnp.int32)
    x1 = jnp.maximum(p - _nmin4(p), 0.0)
    xb1_ref[0] = lax.bitcast_convert_type(x1, jnp.int32)
    stats_ref[0, i] = jnp.max(p) - mn
    stats_ref[1, i] = jnp.max(x1)
    stats_ref[2, i] = mn

    yt = yt_ref[0]
    e = (yt == ch).astype(jnp.int32)
    n1 = jnp.sum(e)
    c0 = jnp.where(n1 >= _NPIX, 0, jnp.minimum(n1, _K))
    nbmin = _nmin4(e)
    eb = e * (1 - nbmin)
    c1 = jnp.minimum(jnp.sum(eb), _K)
    cnt_ref[0, i] = c0
    cnt_ref[1, i] = c1


def _prep(y_pred_softmax, y_true):
    return pl.pallas_call(
        _prep_body,
        grid=(_NIMG,),
        in_specs=[
            pl.BlockSpec((1, 1, _H, _W), lambda i: (i // 3, i % 3 + 1, 0, 0)),
            pl.BlockSpec((1, _H, _W), lambda i: (i // 3, 0, 0)),
        ],
        out_specs=[
            pl.BlockSpec((1, _H, _W), lambda i: (i, 0, 0)),
            pl.BlockSpec((1, _H, _W), lambda i: (i, 0, 0)),
            pl.BlockSpec(memory_space=pltpu.SMEM),
            pl.BlockSpec(memory_space=pltpu.SMEM),
        ],
        out_shape=[
            jax.ShapeDtypeStruct((_NIMG, _H, _W), jnp.int32),
            jax.ShapeDtypeStruct((_NIMG, _H, _W), jnp.int32),
            jax.ShapeDtypeStruct((3, _NIMG), jnp.float32),
            jax.ShapeDtypeStruct((2, _NIMG), jnp.int32),
        ],
    )(y_pred_softmax, y_true)


def _uloop(n, step, body):
    def b(j, carry):
        for u in range(step):
            carry = body(j * step + u, carry)
        return carry

    return lax.fori_loop(0, n // step, b, 0)


def _zero(h, nb):
    z = jnp.zeros((16,), jnp.int32)

    def zb(j, _):
        h[pl.ds(j * 16, 16)] = z
        return 0

    _uloop(nb // 16, 4, zb)


def _swizzle(b, rlog):
    return ((b & ((1 << rlog) - 1)) << 4) | (b >> rlog)


def _scan1(chunk, h):
    ones = jnp.ones((16,), jnp.int32)

    def b1(j, _):
        x = chunk[pl.ds(j * 16, 16)]
        plsc.addupdate_scatter(h, [_swizzle(x >> 20, 7)], ones)
        return 0

    _uloop(_NVEC, 4, b1)


def _scan_dual(chunk, h, keysh, key_k, key_c, bsh, bmask, rlog, coff):
    ones = jnp.ones((16,), jnp.int32)

    def b_dual(j, _):
        x = chunk[pl.ds(j * 16, 16)]
        key = x >> keysh
        addr = _swizzle((x >> bsh) & bmask, rlog)
        plsc.addupdate_scatter(h, [addr], ones, mask=key == key_k)
        plsc.addupdate_scatter(h, [addr + coff], ones, mask=key == key_c)
        return 0

    def b_single(j, _):
        x = chunk[pl.ds(j * 16, 16)]
        addr = _swizzle((x >> bsh) & bmask, rlog)
        plsc.addupdate_scatter(h, [addr], ones, mask=(x >> keysh) == key_k)
        return 0

    eq = key_k == key_c

    @pl.when(eq)
    def _():
        _uloop(_NVEC, 4, b_single)

    @pl.when(jnp.logical_not(eq))
    def _():
        _uloop(_NVEC, 4, b_dual)


def _hsum(v):
    acc = v[0]
    for l in range(1, 16):
        acc = acc + v[l]
    return acc


def _combine(h, stage, scomb, part, tmp, comb, g, t, nb):
    sl = nb // _GRP
    pltpu.sync_copy(h.at[pl.ds(0, nb)], stage.at[g, t, pl.ds(0, nb)])
    plsc.subcore_barrier()
    pltpu.sync_copy(stage.at[g, :, pl.ds(t * sl, sl)], part.at[:, pl.ds(0, sl)])

    def cb(jv, _):
        acc = part[0, pl.ds(jv * 16, 16)]
        for rr in range(1, _GRP):
            acc = acc + part[rr, pl.ds(jv * 16, 16)]
        tmp[pl.ds(jv * 16, 16)] = acc
        return 0

    _uloop(sl // 16, 2, cb)
    pltpu.sync_copy(tmp.at[pl.ds(0, sl)], scomb.at[g, pl.ds(t * sl, sl)])
    plsc.subcore_barrier()
    pltpu.sync_copy(scomb.at[g, pl.ds(0, nb)], comb.at[pl.ds(0, nb)])


def _totals(comb, nb, off):
    rows = nb // 16
    z = jnp.zeros((16,), jnp.int32)
    iota = lax.iota(jnp.int32, 16)

    def tb(j, acc):
        return acc + comb[pl.ds(off + j * 16, 16)]

    def tb4(j, acc):
        for u in range(4):
            acc = acc + comb[pl.ds(off + (j * 4 + u) * 16, 16)]
        return acc

    tvec = lax.fori_loop(0, rows // 4, tb4, z)
    total = _hsum(tvec)
    base = z
    acc = jnp.int32(0)
    for l in range(16):
        base = jnp.where(iota == l, acc, base)
        acc = acc + tvec[l]
    return total, base


def _rank_pass(comb, nb, off, total, base, r):
    rows = nb // 16
    z = jnp.zeros((16,), jnp.int32)
    thr = total - r

    def sb(j, carry):
        pref, cnt, ele = carry
        v = comb[pl.ds(off + j * 16, 16)]
        m = base + pref <= thr
        cnt = cnt + jnp.where(m, 1, 0)
        ele = ele + jnp.where(m, v, 0)
        return pref + v, cnt, ele

    def sb2(j, carry):
        for u in range(2):
            carry = sb(j * 2 + u, carry)
        return carry

    _, cntv, elev = lax.fori_loop(0, rows // 2, sb2, (z, z, z))
    b = _hsum(cntv) - 1
    A = total - _hsum(elev)
    return b, A


def _sc_body(xb0, xb1, rk, th, chunk, rowv, hist, comb, part, tmp, stage,
             scomb):
    c = lax.axis_index("c")
    s = lax.axis_index("s")
    g = s // _GRP
    t = s % _GRP

    def round_body(rnd, _):
        img = rnd * 2 + g
        a = c * _NIMG + img

        @pl.when(c == 0)
        def _():
            pltpu.sync_copy(xb0.at[img, pl.ds(t * _CHUNK, _CHUNK)], chunk)

        @pl.when(c == 1)
        def _():
            pltpu.sync_copy(xb1.at[img, pl.ds(t * _CHUNK, _CHUNK)], chunk)

        pltpu.sync_copy(rk.at[a], rowv)
        r_c = rowv[...][0]

        _zero(hist, _NB1)
        _scan1(chunk, hist)
        _combine(hist, stage, scomb, part, tmp, comb, g, t, _NB1)
        tot1, base1 = _totals(comb, _NB1, 0)
        bk, ak = _rank_pass(comb, _NB1, 0, tot1, base1, jnp.int32(_K))
        bc, ac = _rank_pass(comb, _NB1, 0, tot1, base1, r_c)

        _zero(hist, 2 * _NB2)
        _scan_dual(chunk, hist, 20, bk, bc, 9, _NB2 - 1, 7, _NB2)
        _combine(hist, stage, scomb, part, tmp, comb, g, t, 2 * _NB2)
        off2 = jnp.where(bk == bc, 0, _NB2)
        tk2, bsk2 = _totals(comb, _NB2, 0)
        b2k, a2k = _rank_pass(comb, _NB2, 0, tk2, bsk2, _K - ak)
        tc2, bsc2 = _totals(comb, _NB2, off2)
        b2c, a2c = _rank_pass(comb, _NB2, off2, tc2, bsc2, r_c - ac)

        key3k = (bk << 11) | b2k
        key3c = (bc << 11) | b2c
        _zero(hist, 2 * _NB3)
        _scan_dual(chunk, hist, 9, key3k, key3c, 0, _NB3 - 1, 5, _NB3)
        _combine(hist, stage, scomb, part, tmp, comb, g, t, 2 * _NB3)
        off3 = jnp.where(key3k == key3c, 0, _NB3)
        tk3, bsk3 = _totals(comb, _NB3, 0)
        b3k, _u = _rank_pass(comb, _NB3, 0, tk3, bsk3, _K - ak - a2k)
        tc3, bsc3 = _totals(comb, _NB3, off3)
        b3c, _u2 = _rank_pass(comb, _NB3, off3, tc3, bsc3, r_c - ac - a2c)

        lo_k = (bk << 20) | (b2k << 9) | b3k
        lo_c = (bc << 20) | (b2c << 9) | b3c

        @pl.when(t == 0)
        def _():
            iota = lax.iota(jnp.int32, 16)
            rowv[...] = jnp.where(iota == 0, lo_k,
                                  jnp.where(iota == 1, lo_c, 0))
            pltpu.sync_copy(rowv, th.at[a])

        return 0

    lax.fori_loop(0, _NIMG // 2, round_body, 0)


_sc_select = functools.partial(
    pl.kernel,
    out_type=jax.ShapeDtypeStruct((2 * _NIMG, 16), jnp.int32),
    mesh=plsc.VectorSubcoreMesh(core_axis_name="c", subcore_axis_name="s"),
    scratch_types=[
        pltpu.VMEM((_CHUNK,), jnp.int32),
        pltpu.VMEM((16,), jnp.int32),
        pltpu.VMEM((2 * _NB2,), jnp.int32),
        pltpu.VMEM((2 * _NB2,), jnp.int32),
        pltpu.VMEM((_GRP, 2 * _NB2 // _GRP), jnp.int32),
        pltpu.VMEM((2 * _NB2 // _GRP,), jnp.int32),
        pltpu.VMEM_SHARED((2, _GRP, 2 * _NB2), jnp.int32),
        pltpu.VMEM_SHARED((2, 2 * _NB2), jnp.int32),
    ],
    compiler_params=pltpu.CompilerParams(needs_layout_passes=False),
)(_sc_body)


def _final_body(p_ref, stats_ref, cnt_ref, th_ref, out_ref, acc_ref):
    i = pl.program_id(0)
    p = p_ref[0, 0]

    @pl.when(i == 0)
    def _init():
        acc_ref[0] = 0.0
        acc_ref[1] = 0.0

    for v in (0, 1):
        if v == 0:
            x = p - stats_ref[2, i]
        else:
            x = jnp.maximum(p - _nmin4(p), 0.0)
        gmax = stats_ref[v, 0]
        for j in range(1, _NIMG):
            gmax = jnp.maximum(gmax, stats_ref[v, j])
        g = jnp.maximum(gmax, 1e-8)
        tau = _TH * g
        c = cnt_ref[v, i]
        tk = _f32_val(th_ref[v * _NIMG + i, 0])
        tc = _f32_val(th_ref[v * _NIMG + i, 1])

        m2 = jnp.maximum(tk, tau)
        m1 = jnp.maximum(tc, tau)
        mask2 = x > m2
        mask1 = x > m1
        a2 = jnp.sum(jnp.where(mask2, x * x, 0.0))
        c2 = jnp.sum(mask2.astype(jnp.int32)).astype(jnp.float32)
        a1 = jnp.sum(jnp.where(mask1, x, 0.0))
        c1n = jnp.sum(mask1.astype(jnp.int32)).astype(jnp.float32)
        cf = c.astype(jnp.float32)
        s2 = (a2 + jnp.where(tk > tau, (_K - c2) * tk * tk, 0.0)) / (g * g)
        t_sum = jnp.where(
            c > 0,
            (a1 + jnp.where(tc > tau, (cf - c1n) * tc, 0.0)) / g,
            0.0)
        acc_ref[v] = acc_ref[v] + s2 - 2.0 * t_sum + cf

    @pl.when(i == _NIMG - 1)
    def _final():
        l0 = acc_ref[0] / (_NIMG * _K)
        l1 = acc_ref[1] / (_NIMG * _K)
        den = jnp.maximum(l0 + l1, 1e-8)
        out_ref[...] = jnp.full((1, 1), 2.0 * l0 * l1 / den, jnp.float32)


def _final(y_pred_softmax, stats, cnt, th):
    return pl.pallas_call(
        _final_body,
        grid=(_NIMG,),
        in_specs=[
            pl.BlockSpec((1, 1, _H, _W), lambda i: (i // 3, i % 3 + 1, 0, 0)),
            pl.BlockSpec(memory_space=pltpu.SMEM),
            pl.BlockSpec(memory_space=pltpu.SMEM),
            pl.BlockSpec(memory_space=pltpu.SMEM),
        ],
        out_specs=pl.BlockSpec((1, 1), lambda i: (0, 0)),
        out_shape=jax.ShapeDtypeStruct((1, 1), jnp.float32),
        scratch_shapes=[pltpu.SMEM((2,), jnp.float32)],
    )(y_pred_softmax, stats, cnt, th)


def kernel(y_pred_softmax, y_true):
    xb0, xb1, stats, cnt = _prep(y_pred_softmax, y_true)
    ranks = jnp.clip(cnt, 1, _K).reshape(2 * _NIMG).astype(jnp.int32)
    rank_pad = jnp.zeros((2 * _NIMG, 16), jnp.int32).at[:, 0].set(ranks)
    th = _sc_select(xb0.reshape(_NIMG, _NPIX), xb1.reshape(_NIMG, _NPIX),
                    rank_pad)
    return _final(y_pred_softmax, stats, cnt, th)[0, 0]


# speedup vs baseline: 1.1580x; 1.0103x over previous
import functools

import jax
import jax.numpy as jnp
from jax import lax
from jax.experimental import pallas as pl
from jax.experimental.pallas import tpu as pltpu
from jax.experimental.pallas import tpu_sc as plsc

_K = 2000
_TH = 0.001
_NIMG = 12
_H = 512
_W = 512
_NPIX = _H * _W

_GRP = 8
_CHUNK = _NPIX // _GRP
_NVEC = _CHUNK // 16
_NB1 = 2048
_NB2 = 2048
_NB3 = 512


def _f32_val(xi):
    v = jnp.full((8, 128), xi, jnp.int32)
    return jnp.max(lax.bitcast_convert_type(v, jnp.float32))


def _nmin4(x):
    a = pltpu.roll(x, 1, 0)
    b = pltpu.roll(x, _H - 1, 0)
    c = pltpu.roll(x, 1, 1)
    d = pltpu.roll(x, _W - 1, 1)
    return jnp.minimum(jnp.minimum(a, b), jnp.minimum(c, d))


def _prep_body(p_ref, yt_ref, xb0_ref, xb1_ref, stats_ref, cnt_ref):
    i = pl.program_id(0)
    ch = i % 3 + 1
    p = p_ref[0, 0]
    mn = jnp.min(p)
    x0 = p - mn
    xb0_ref[0] = lax.bitcast_convert_type(x0, jnp.int32)
    x1 = jnp.maximum(p - _nmin4(p), 0.0)
    xb1_ref[0] = lax.bitcast_convert_type(x1, jnp.int32)
    stats_ref[0, i] = jnp.max(p) - mn
    stats_ref[1, i] = jnp.max(x1)
    stats_ref[2, i] = mn

    yt = yt_ref[0]
    e = (yt == ch).astype(jnp.int32)
    n1 = jnp.sum(e)
    c0 = jnp.where(n1 >= _NPIX, 0, jnp.minimum(n1, _K))
    nbmin = _nmin4(e)
    eb = e * (1 - nbmin)
    c1 = jnp.minimum(jnp.sum(eb), _K)
    cnt_ref[0, i] = c0
    cnt_ref[1, i] = c1


def _prep(y_pred_softmax, y_true):
    return pl.pallas_call(
        _prep_body,
        grid=(_NIMG,),
        in_specs=[
            pl.BlockSpec((1, 1, _H, _W), lambda i: (i // 3, i % 3 + 1, 0, 0)),
            pl.BlockSpec((1, _H, _W), lambda i: (i // 3, 0, 0)),
        ],
        out_specs=[
            pl.BlockSpec((1, _H, _W), lambda i: (i, 0, 0)),
            pl.BlockSpec((1, _H, _W), lambda i: (i, 0, 0)),
            pl.BlockSpec(memory_space=pltpu.SMEM),
            pl.BlockSpec(memory_space=pltpu.SMEM),
        ],
        out_shape=[
            jax.ShapeDtypeStruct((_NIMG, _H, _W), jnp.int32),
            jax.ShapeDtypeStruct((_NIMG, _H, _W), jnp.int32),
            jax.ShapeDtypeStruct((3, _NIMG), jnp.float32),
            jax.ShapeDtypeStruct((2, _NIMG), jnp.int32),
        ],
    )(y_pred_softmax, y_true)


def _uloop(n, step, body):
    def b(j, carry):
        for u in range(step):
            carry = body(j * step + u, carry)
        return carry

    return lax.fori_loop(0, n // step, b, 0)


def _zero(h, nb):
    z = jnp.zeros((16,), jnp.int32)

    def zb(j, _):
        h[pl.ds(j * 16, 16)] = z
        return 0

    _uloop(nb // 16, 4, zb)


def _swizzle(b, rlog):
    return ((b & ((1 << rlog) - 1)) << 4) | (b >> rlog)


def _scan1(chunk, h):
    ones = jnp.ones((16,), jnp.int32)

    def b1(j, _):
        x = chunk[pl.ds(j * 16, 16)]
        plsc.addupdate_scatter(h, [_swizzle(x >> 20, 7)], ones)
        return 0

    _uloop(_NVEC, 8, b1)


def _scan_dual(chunk, h, keysh, key_k, key_c, bsh, bmask, rlog, coff):
    ones = jnp.ones((16,), jnp.int32)

    def b_dual(j, _):
        x = chunk[pl.ds(j * 16, 16)]
        key = x >> keysh
        addr = _swizzle((x >> bsh) & bmask, rlog)
        plsc.addupdate_scatter(h, [addr], ones, mask=key == key_k)
        plsc.addupdate_scatter(h, [addr + coff], ones, mask=key == key_c)
        return 0

    def b_single(j, _):
        x = chunk[pl.ds(j * 16, 16)]
        addr = _swizzle((x >> bsh) & bmask, rlog)
        plsc.addupdate_scatter(h, [addr], ones, mask=(x >> keysh) == key_k)
        return 0

    eq = key_k == key_c

    @pl.when(eq)
    def _():
        _uloop(_NVEC, 8, b_single)

    @pl.when(jnp.logical_not(eq))
    def _():
        _uloop(_NVEC, 8, b_dual)


def _hsum(v):
    acc = v[0]
    for l in range(1, 16):
        acc = acc + v[l]
    return acc


def _combine(h, stage, scomb, part, tmp, comb, g, t, nb):
    sl = nb // _GRP
    pltpu.sync_copy(h.at[pl.ds(0, nb)], stage.at[g, t, pl.ds(0, nb)])
    plsc.subcore_barrier()
    pltpu.sync_copy(stage.at[g, :, pl.ds(t * sl, sl)], part.at[:, pl.ds(0, sl)])

    def cb(jv, _):
        acc = part[0, pl.ds(jv * 16, 16)]
        for rr in range(1, _GRP):
            acc = acc + part[rr, pl.ds(jv * 16, 16)]
        tmp[pl.ds(jv * 16, 16)] = acc
        return 0

    _uloop(sl // 16, 2, cb)
    pltpu.sync_copy(tmp.at[pl.ds(0, sl)], scomb.at[g, pl.ds(t * sl, sl)])
    plsc.subcore_barrier()
    pltpu.sync_copy(scomb.at[g, pl.ds(0, nb)], comb.at[pl.ds(0, nb)])


def _totals(comb, nb, off):
    rows = nb // 16
    z = jnp.zeros((16,), jnp.int32)
    iota = lax.iota(jnp.int32, 16)

    def tb(j, acc):
        return acc + comb[pl.ds(off + j * 16, 16)]

    def tb4(j, acc):
        for u in range(4):
            acc = acc + comb[pl.ds(off + (j * 4 + u) * 16, 16)]
        return acc

    tvec = lax.fori_loop(0, rows // 4, tb4, z)
    total = _hsum(tvec)
    base = z
    acc = jnp.int32(0)
    for l in range(16):
        base = jnp.where(iota == l, acc, base)
        acc = acc + tvec[l]
    return total, base


def _rank_pass(comb, nb, off, total, base, r):
    rows = nb // 16
    z = jnp.zeros((16,), jnp.int32)
    thr = total - r

    def sb(j, carry):
        pref, cnt, ele = carry
        v = comb[pl.ds(off + j * 16, 16)]
        m = base + pref <= thr
        cnt = cnt + jnp.where(m, 1, 0)
        ele = ele + jnp.where(m, v, 0)
        return pref + v, cnt, ele

    def sb2(j, carry):
        for u in range(2):
            carry = sb(j * 2 + u, carry)
        return carry

    _, cntv, elev = lax.fori_loop(0, rows // 2, sb2, (z, z, z))
    b = _hsum(cntv) - 1
    A = total - _hsum(elev)
    return b, A


def _sc_body(xb0, xb1, rk, th, chunk, rowv, hist, comb, part, tmp, stage,
             scomb):
    c = lax.axis_index("c")
    s = lax.axis_index("s")
    g = s // _GRP
    t = s % _GRP

    def round_body(rnd, _):
        img = rnd * 2 + g
        a = c * _NIMG + img

        @pl.when(c == 0)
        def _():
            pltpu.sync_copy(xb0.at[img, pl.ds(t * _CHUNK, _CHUNK)], chunk)

        @pl.when(c == 1)
        def _():
            pltpu.sync_copy(xb1.at[img, pl.ds(t * _CHUNK, _CHUNK)], chunk)

        pltpu.sync_copy(rk.at[a], rowv)
        r_c = rowv[...][0]

        _zero(hist, _NB1)
        _scan1(chunk, hist)
        _combine(hist, stage, scomb, part, tmp, comb, g, t, _NB1)
        tot1, base1 = _totals(comb, _NB1, 0)
        bk, ak = _rank_pass(comb, _NB1, 0, tot1, base1, jnp.int32(_K))
        bc, ac = _rank_pass(comb, _NB1, 0, tot1, base1, r_c)

        _zero(hist, 2 * _NB2)
        _scan_dual(chunk, hist, 20, bk, bc, 9, _NB2 - 1, 7, _NB2)
        _combine(hist, stage, scomb, part, tmp, comb, g, t, 2 * _NB2)
        off2 = jnp.where(bk == bc, 0, _NB2)
        tk2, bsk2 = _totals(comb, _NB2, 0)
        b2k, a2k = _rank_pass(comb, _NB2, 0, tk2, bsk2, _K - ak)
        tc2, bsc2 = _totals(comb, _NB2, off2)
        b2c, a2c = _rank_pass(comb, _NB2, off2, tc2, bsc2, r_c - ac)

        key3k = (bk << 11) | b2k
        key3c = (bc << 11) | b2c
        _zero(hist, 2 * _NB3)
        _scan_dual(chunk, hist, 9, key3k, key3c, 0, _NB3 - 1, 5, _NB3)
        _combine(hist, stage, scomb, part, tmp, comb, g, t, 2 * _NB3)
        off3 = jnp.where(key3k == key3c, 0, _NB3)
        tk3, bsk3 = _totals(comb, _NB3, 0)
        b3k, _u = _rank_pass(comb, _NB3, 0, tk3, bsk3, _K - ak - a2k)
        tc3, bsc3 = _totals(comb, _NB3, off3)
        b3c, _u2 = _rank_pass(comb, _NB3, off3, tc3, bsc3, r_c - ac - a2c)

        lo_k = (bk << 20) | (b2k << 9) | b3k
        lo_c = (bc << 20) | (b2c << 9) | b3c

        @pl.when(t == 0)
        def _():
            iota = lax.iota(jnp.int32, 16)
            rowv[...] = jnp.where(iota == 0, lo_k,
                                  jnp.where(iota == 1, lo_c, 0))
            pltpu.sync_copy(rowv, th.at[a])

        return 0

    lax.fori_loop(0, _NIMG // 2, round_body, 0)


_sc_select = functools.partial(
    pl.kernel,
    out_type=jax.ShapeDtypeStruct((2 * _NIMG, 16), jnp.int32),
    mesh=plsc.VectorSubcoreMesh(core_axis_name="c", subcore_axis_name="s"),
    scratch_types=[
        pltpu.VMEM((_CHUNK,), jnp.int32),
        pltpu.VMEM((16,), jnp.int32),
        pltpu.VMEM((2 * _NB2,), jnp.int32),
        pltpu.VMEM((2 * _NB2,), jnp.int32),
        pltpu.VMEM((_GRP, 2 * _NB2 // _GRP), jnp.int32),
        pltpu.VMEM((2 * _NB2 // _GRP,), jnp.int32),
        pltpu.VMEM_SHARED((2, _GRP, 2 * _NB2), jnp.int32),
        pltpu.VMEM_SHARED((2, 2 * _NB2), jnp.int32),
    ],
    compiler_params=pltpu.CompilerParams(needs_layout_passes=False),
)(_sc_body)


def _final_body(p_ref, stats_ref, cnt_ref, th_ref, out_ref, acc_ref):
    i = pl.program_id(0)
    p = p_ref[0, 0]

    @pl.when(i == 0)
    def _init():
        acc_ref[0] = 0.0
        acc_ref[1] = 0.0

    for v in (0, 1):
        if v == 0:
            x = p - stats_ref[2, i]
        else:
            x = jnp.maximum(p - _nmin4(p), 0.0)
        gmax = stats_ref[v, 0]
        for j in range(1, _NIMG):
            gmax = jnp.maximum(gmax, stats_ref[v, j])
        g = jnp.maximum(gmax, 1e-8)
        tau = _TH * g
        c = cnt_ref[v, i]
        tk = _f32_val(th_ref[v * _NIMG + i, 0])
        tc = _f32_val(th_ref[v * _NIMG + i, 1])

        m2 = jnp.maximum(tk, tau)
        m1 = jnp.maximum(tc, tau)
        mask2 = x > m2
        mask1 = x > m1
        a2 = jnp.sum(jnp.where(mask2, x * x, 0.0))
        c2 = jnp.sum(mask2.astype(jnp.int32)).astype(jnp.float32)
        a1 = jnp.sum(jnp.where(mask1, x, 0.0))
        c1n = jnp.sum(mask1.astype(jnp.int32)).astype(jnp.float32)
        cf = c.astype(jnp.float32)
        s2 = (a2 + jnp.where(tk > tau, (_K - c2) * tk * tk, 0.0)) / (g * g)
        t_sum = jnp.where(
            c > 0,
            (a1 + jnp.where(tc > tau, (cf - c1n) * tc, 0.0)) / g,
            0.0)
        acc_ref[v] = acc_ref[v] + s2 - 2.0 * t_sum + cf

    @pl.when(i == _NIMG - 1)
    def _final():
        l0 = acc_ref[0] / (_NIMG * _K)
        l1 = acc_ref[1] / (_NIMG * _K)
        den = jnp.maximum(l0 + l1, 1e-8)
        out_ref[...] = jnp.full((1, 1), 2.0 * l0 * l1 / den, jnp.float32)


def _final(y_pred_softmax, stats, cnt, th):
    return pl.pallas_call(
        _final_body,
        grid=(_NIMG,),
        in_specs=[
            pl.BlockSpec((1, 1, _H, _W), lambda i: (i // 3, i % 3 + 1, 0, 0)),
            pl.BlockSpec(memory_space=pltpu.SMEM),
            pl.BlockSpec(memory_space=pltpu.SMEM),
            pl.BlockSpec(memory_space=pltpu.SMEM),
        ],
        out_specs=pl.BlockSpec((1, 1), lambda i: (0, 0)),
        out_shape=jax.ShapeDtypeStruct((1, 1), jnp.float32),
        scratch_shapes=[pltpu.SMEM((2,), jnp.float32)],
    )(y_pred_softmax, stats, cnt, th)


def kernel(y_pred_softmax, y_true):
    xb0, xb1, stats, cnt = _prep(y_pred_softmax, y_true)
    ranks = jnp.clip(cnt, 1, _K).reshape(2 * _NIMG).astype(jnp.int32)
    rank_pad = jnp.zeros((2 * _NIMG, 16), jnp.int32).at[:, 0].set(ranks)
    th = _sc_select(xb0.reshape(_NIMG, _NPIX), xb1.reshape(_NIMG, _NPIX),
                    rank_pad)
    return _final(y_pred_softmax, stats, cnt, th)[0, 0]
